# baseline (speedup 1.0000x reference)
import functools

import jax
import jax.numpy as jnp
from jax import lax
from jax.experimental import pallas as pl
from jax.experimental.pallas import tpu as pltpu
from jax.experimental.pallas import tpu_sc as plsc

DIM = 128
NLANE = 16
NVEC = DIM // NLANE
NC, NS = 2, 16
NW = NC * NS
CB = 8
GSZ = 80


def _sc_scores(B, K, comb, W_v, W_u):
    BPW = B // NW
    NCHUNK = BPW // CB
    ROWS = CB * K
    NG = ROWS // GSZ
    IDXW = 2 * CB + ROWS
    assert B % NW == 0 and BPW % CB == 0 and ROWS % GSZ == 0

    mesh = plsc.VectorSubcoreMesh(
        core_axis_name="c", subcore_axis_name="s", num_cores=NC, num_subcores=NS
    )

    @functools.partial(
        pl.kernel,
        out_type=(
            jax.ShapeDtypeStruct((B * NLANE,), jnp.float32),
            jax.ShapeDtypeStruct((B * NLANE,), jnp.float32),
        ),
        mesh=mesh,
        scratch_types=dict(
            ibuf=[pltpu.VMEM((IDXW,), jnp.int32)] * 2,
            cbuf=[pltpu.VMEM((CB, DIM), jnp.float32)] * 2,
            tbuf=[pltpu.VMEM((CB, DIM), jnp.float32)] * 2,
            nbuf=[pltpu.VMEM((ROWS, DIM), jnp.float32)] * 2,
            posb=pltpu.VMEM((BPW * NLANE,), jnp.float32),
            negb=pltpu.VMEM((BPW * NLANE,), jnp.float32),
            rsem=[pltpu.SemaphoreType.DMA] * 2,
            isem=[pltpu.SemaphoreType.DMA] * 2,
        ),
    )
    def sc_kernel(comb_h, wv_h, wu_h, pos_h, neg_h, *,
                  ibuf, cbuf, tbuf, nbuf, posb, negb,
                  rsem, isem):
        wid = lax.axis_index("s") * NC + lax.axis_index("c")
        wbase = wid * BPW

        def idx_copy(ci, slot):
            base = (wbase // CB + ci) * IDXW
            return pltpu.make_async_copy(
                comb_h.at[pl.ds(base, IDXW)], ibuf[slot], isem[slot])

        def fire_idx(ci, slot):
            idx_copy(ci, slot).start()

        def wait_idx(ci, slot):
            idx_copy(ci, slot).wait()

        def row_copies(slot):
            ds = [
                pltpu.make_async_copy(
                    wv_h.at[ibuf[slot].at[pl.ds(0, CB)]], cbuf[slot], rsem[slot]),
                pltpu.make_async_copy(
                    wu_h.at[ibuf[slot].at[pl.ds(CB, CB)]], tbuf[slot], rsem[slot]),
            ]
            for g in range(NG):
                ds.append(pltpu.make_async_copy(
                    wu_h.at[ibuf[slot].at[pl.ds(2 * CB + g * GSZ, GSZ)]],
                    nbuf[slot].at[pl.ds(g * GSZ, GSZ)],
                    rsem[slot],
                ))
            return ds

        def fire_rows(slot):
            for d in row_copies(slot):
                d.start()

        def drain_rows(slot):
            for d in row_copies(slot):
                d.wait()

        def compute(ci, slot):
            cb, tb, nb = cbuf[slot], tbuf[slot], nbuf[slot]
            for b in range(CB):
                cvecs = [cb[b, pl.ds(j * NLANE, NLANE)] for j in range(NVEC)]
                pacc = tb[b, pl.ds(0, NLANE)] * cvecs[0]
                for j in range(1, NVEC):
                    pacc = pacc + tb[b, pl.ds(j * NLANE, NLANE)] * cvecs[j]

                def kbody(k, accs):
                    r = b * K + k
                    return tuple(
                        accs[j] + nb[r, pl.ds(j * NLANE, NLANE)] * cvecs[j]
                        for j in range(NVEC)
                    )
                zero = jnp.zeros((NLANE,), jnp.float32)
                naccs = lax.fori_loop(0, K, kbody, (zero,) * NVEC, unroll=2)
                nacc = naccs[0]
                for j in range(1, NVEC):
                    nacc = nacc + naccs[j]

                bl = ci * CB + b
                posb[pl.ds(bl * NLANE, NLANE)] = pacc
                negb[pl.ds(bl * NLANE, NLANE)] = nacc

        fire_idx(0, 0)
        wait_idx(0, 0)
        fire_rows(0)
        fire_idx(1, 1)
        wait_idx(1, 1)

        @pl.loop(0, NCHUNK, step=2)
        def _(i):
            fire_rows(1)
            drain_rows(0)
            @pl.when(i + 2 < NCHUNK)
            def _():
                fire_idx(i + 2, 0)
            compute(i, 0)
            @pl.when(i + 2 < NCHUNK)
            def _():
                wait_idx(i + 2, 0)
                fire_rows(0)
            drain_rows(1)
            @pl.when(i + 3 < NCHUNK)
            def _():
                fire_idx(i + 3, 1)
            compute(i + 1, 1)
            @pl.when(i + 3 < NCHUNK)
            def _():
                wait_idx(i + 3, 1)

        pltpu.sync_copy(posb, pos_h.at[pl.ds(wbase * NLANE, BPW * NLANE)])
        pltpu.sync_copy(negb, neg_h.at[pl.ds(wbase * NLANE, BPW * NLANE)])

    return sc_kernel(comb, W_v, W_u)


def _loss_body(pos_ref, neg_ref, out_ref, *, n):
    seg = (lax.broadcasted_iota(jnp.int32, (DIM, 8), 0) // NLANE
           == lax.broadcasted_iota(jnp.int32, (DIM, 8), 1)).astype(jnp.float32)
    p = jnp.dot(pos_ref[...], seg, preferred_element_type=jnp.float32)
    q = jnp.dot(neg_ref[...], seg, preferred_element_type=jnp.float32)
    ls = jax.nn.log_sigmoid(p) + jax.nn.log_sigmoid(-q)
    out_ref[0, 0] = -jnp.sum(ls) * (1.0 / n)


def kernel(center_words, target_words, negative_words, W_v, W_u):
    B, K = negative_words.shape
    nchunks = B // CB
    comb = jnp.concatenate(
        [center_words.reshape(nchunks, CB),
         target_words.reshape(nchunks, CB),
         negative_words.reshape(nchunks, CB * K)],
        axis=1,
    ).reshape(-1)

    pos, neg = _sc_scores(B, K, comb, W_v, W_u)

    r = B * NLANE // DIM
    out = pl.pallas_call(
        functools.partial(_loss_body, n=B),
        out_shape=jax.ShapeDtypeStruct((1, 1), jnp.float32),
        out_specs=pl.BlockSpec(memory_space=pltpu.SMEM),
    )(pos.reshape(r, DIM), neg.reshape(r, DIM))
    return out[0, 0]


# speedup vs baseline: 1.0488x; 1.0488x over previous
import functools

import jax
import jax.numpy as jnp
from jax import lax
from jax.experimental import pallas as pl
from jax.experimental.pallas import tpu as pltpu
from jax.experimental.pallas import tpu_sc as plsc

DIM = 128
NLANE = 16
NVEC = DIM // NLANE
NC, NS = 2, 16
NW = NC * NS
CB = 8
GSZ = 80


def _sc_scores(B, K, center, target, neg_flat, W_v, W_u):
    BPW = B // NW
    NCHUNK = BPW // CB
    ROWS = CB * K
    NG = ROWS // GSZ
    assert B % NW == 0 and BPW % CB == 0 and ROWS % GSZ == 0

    mesh = plsc.VectorSubcoreMesh(
        core_axis_name="c", subcore_axis_name="s", num_cores=NC, num_subcores=NS
    )

    @functools.partial(
        pl.kernel,
        out_type=(
            jax.ShapeDtypeStruct((B * NLANE,), jnp.float32),
            jax.ShapeDtypeStruct((B * NLANE,), jnp.float32),
        ),
        mesh=mesh,
        scratch_types=dict(
            cidx=[pltpu.VMEM((CB,), jnp.int32)] * 2,
            tidx=[pltpu.VMEM((CB,), jnp.int32)] * 2,
            nidx=[pltpu.VMEM((ROWS,), jnp.int32)] * 2,
            cbuf=[pltpu.VMEM((CB, DIM), jnp.float32)] * 2,
            tbuf=[pltpu.VMEM((CB, DIM), jnp.float32)] * 2,
            nbuf=[pltpu.VMEM((ROWS, DIM), jnp.float32)] * 2,
            posb=pltpu.VMEM((BPW * NLANE,), jnp.float32),
            negb=pltpu.VMEM((BPW * NLANE,), jnp.float32),
            rsem=[pltpu.SemaphoreType.DMA] * 2,
            isem=[pltpu.SemaphoreType.DMA] * 2,
        ),
    )
    def sc_kernel(center_h, target_h, negf_h, wv_h, wu_h, pos_h, neg_h, *,
                  cidx, tidx, nidx, cbuf, tbuf, nbuf, posb, negb,
                  rsem, isem):
        wid = lax.axis_index("s") * NC + lax.axis_index("c")
        wbase = wid * BPW

        def idx_copies(ci, slot):
            base = wbase + ci * CB
            return (
                pltpu.make_async_copy(center_h.at[pl.ds(base, CB)], cidx[slot], isem[slot]),
                pltpu.make_async_copy(target_h.at[pl.ds(base, CB)], tidx[slot], isem[slot]),
                pltpu.make_async_copy(negf_h.at[pl.ds(base * K, ROWS)], nidx[slot], isem[slot]),
            )

        def fire_idx(ci, slot):
            for d in idx_copies(ci, slot):
                d.start()

        def wait_idx(ci, slot):
            for d in idx_copies(ci, slot):
                d.wait()

        def row_copies(slot):
            ds = [
                pltpu.make_async_copy(wv_h.at[cidx[slot]], cbuf[slot], rsem[slot]),
                pltpu.make_async_copy(wu_h.at[tidx[slot]], tbuf[slot], rsem[slot]),
            ]
            for g in range(NG):
                ds.append(pltpu.make_async_copy(
                    wu_h.at[nidx[slot].at[pl.ds(g * GSZ, GSZ)]],
                    nbuf[slot].at[pl.ds(g * GSZ, GSZ)],
                    rsem[slot],
                ))
            return ds

        def fire_rows(slot):
            for d in row_copies(slot):
                d.start()

        def drain_rows(slot):
            for d in row_copies(slot):
                d.wait()

        def compute(ci, slot):
            cb, tb, nb = cbuf[slot], tbuf[slot], nbuf[slot]
            for b in range(CB):
                cvecs = [cb[b, pl.ds(j * NLANE, NLANE)] for j in range(NVEC)]
                pacc = tb[b, pl.ds(0, NLANE)] * cvecs[0]
                for j in range(1, NVEC):
                    pacc = pacc + tb[b, pl.ds(j * NLANE, NLANE)] * cvecs[j]

                def kbody(k, accs):
                    r = b * K + k
                    return tuple(
                        accs[j] + nb[r, pl.ds(j * NLANE, NLANE)] * cvecs[j]
                        for j in range(NVEC)
                    )
                zero = jnp.zeros((NLANE,), jnp.float32)
                naccs = lax.fori_loop(0, K, kbody, (zero,) * NVEC, unroll=2)
                nacc = naccs[0]
                for j in range(1, NVEC):
                    nacc = nacc + naccs[j]

                bl = ci * CB + b
                posb[pl.ds(bl * NLANE, NLANE)] = pacc
                negb[pl.ds(bl * NLANE, NLANE)] = nacc

        fire_idx(0, 0)
        wait_idx(0, 0)
        fire_rows(0)
        fire_idx(1, 1)
        wait_idx(1, 1)

        @pl.loop(0, NCHUNK, step=2)
        def _(i):
            fire_rows(1)
            drain_rows(0)
            @pl.when(i + 2 < NCHUNK)
            def _():
                fire_idx(i + 2, 0)
            compute(i, 0)
            @pl.when(i + 2 < NCHUNK)
            def _():
                wait_idx(i + 2, 0)
                fire_rows(0)
            drain_rows(1)
            @pl.when(i + 3 < NCHUNK)
            def _():
                fire_idx(i + 3, 1)
            compute(i + 1, 1)
            @pl.when(i + 3 < NCHUNK)
            def _():
                wait_idx(i + 3, 1)

        pltpu.sync_copy(posb, pos_h.at[pl.ds(wbase * NLANE, BPW * NLANE)])
        pltpu.sync_copy(negb, neg_h.at[pl.ds(wbase * NLANE, BPW * NLANE)])

    return sc_kernel(center, target, neg_flat, W_v, W_u)


def _loss_body(pos_ref, neg_ref, out_ref, *, n):
    seg = (lax.broadcasted_iota(jnp.int32, (DIM, 8), 0) // NLANE
           == lax.broadcasted_iota(jnp.int32, (DIM, 8), 1)).astype(jnp.float32)
    p = jnp.dot(pos_ref[...], seg, preferred_element_type=jnp.float32)
    q = jnp.dot(neg_ref[...], seg, preferred_element_type=jnp.float32)
    ls = jax.nn.log_sigmoid(p) + jax.nn.log_sigmoid(-q)
    out_ref[0, 0] = -jnp.sum(ls) * (1.0 / n)


def kernel(center_words, target_words, negative_words, W_v, W_u):
    B, K = negative_words.shape
    center = center_words.reshape(B)
    target = target_words.reshape(B)
    neg_flat = negative_words.reshape(B * K)

    pos, neg = _sc_scores(B, K, center, target, neg_flat, W_v, W_u)

    r = B * NLANE // DIM
    out = pl.pallas_call(
        functools.partial(_loss_body, n=B),
        out_shape=jax.ShapeDtypeStruct((1, 1), jnp.float32),
        out_specs=pl.BlockSpec(memory_space=pltpu.SMEM),
    )(pos.reshape(r, DIM), neg.reshape(r, DIM))
    return out[0, 0]
